# baseline (speedup 1.0000x reference)
import math
import jax, jax.numpy as jnp
from jax.experimental import pallas as pl

N = 10000
E = 160000
RATIO = 0.5

def _bn(x, g, b):
    m = jnp.mean(x, axis=0)
    v = jnp.var(x, axis=0)
    return (x - m) / jnp.sqrt(v + 1e-5) * g + b

def _topk_pool(x, w, n, ratio):
    score = x @ w / jnp.linalg.norm(w)
    k = int(math.ceil(ratio * n))
    _, perm = jax.lax.top_k(score, k)
    xp = x[perm] * jnp.tanh(score[perm])[:, None]
    new_id = jnp.full((n,), -1, dtype=jnp.int32).at[perm].set(jnp.arange(k, dtype=jnp.int32))
    return xp, new_id, k

def kernel(node, edge_index, batch, W1_rel, b1_rel, W1_root, bn1_g, bn1_b, pool1_w, W2_rel, b2_rel, W2_root, bn2_g, bn2_b, pool2_w, Wfc, bfc, Wfc1, bfc1):
    src, dst = edge_index[0], edge_index[1]
    agg = jax.ops.segment_sum(node[src], dst, num_segments=N)
    x = agg @ W1_rel + b1_rel + node @ W1_root
    x = _bn(x, bn1_g, bn1_b)
    x = jax.nn.relu(x)
    x, new_id, k1 = _topk_pool(x, pool1_w, N, RATIO)
    src1 = new_id[src]
    dst1 = new_id[dst]
    keep = (src1 >= 0) & (dst1 >= 0)
    x1 = jnp.mean(x, axis=0, keepdims=True)
    msg = x[jnp.where(keep, src1, 0)] * keep[:, None].astype(x.dtype)
    agg2 = jax.ops.segment_sum(msg, jnp.where(keep, dst1, 0), num_segments=k1)
    x = agg2 @ W2_rel + b2_rel + x @ W2_root
    x = _bn(x, bn2_g, bn2_b)
    x = jax.nn.relu(x)
    x, _, k2 = _topk_pool(x, pool2_w, k1, RATIO)
    x2 = jnp.mean(x, axis=0, keepdims=True)
    x = x1 + x2
    x = jax.nn.relu(x @ Wfc + bfc)
    out = x @ Wfc1 + bfc1
    return out


# speedup vs baseline: 1.0847x; 1.0847x over previous
import functools
import math

import jax
import jax.numpy as jnp
from jax import lax
from jax.experimental import pallas as pl
from jax.experimental.pallas import tpu as pltpu
from jax.experimental.pallas import tpu_sc as plsc

N = 10000
NP = 10240
E = 160000
F = 128
H = 1024
RATIO = 0.5

NW = 32
EPW = E // NW
CE = 200
NCH = EPW // CE
ZR = 160


def _seg_sum_128_sc(node_pad, src, dst):
    mesh = plsc.VectorSubcoreMesh(core_axis_name="c", subcore_axis_name="s")

    @functools.partial(
        pl.kernel, mesh=mesh,
        out_type=jax.ShapeDtypeStruct((2, NP, F), jnp.float32),
        scratch_types=[
            pltpu.VMEM((CE,), jnp.int32),
            pltpu.VMEM((CE,), jnp.int32),
            pltpu.VMEM((CE, F), jnp.float32),
            pltpu.VMEM((ZR, F), jnp.float32),
            pltpu.VMEM_SHARED((NP, F), jnp.float32),
            pltpu.SemaphoreType.DMA,
        ],
    )
    def k(node_hbm, src_hbm, dst_hbm, out_hbm, sidx, didx, rows, zbuf, acc, sem):
        c = lax.axis_index("c")
        s = lax.axis_index("s")

        def zrow(i, carry):
            def zcol(j, cc):
                zbuf[i, pl.ds(j * 16, 16)] = jnp.zeros((16,), jnp.float32)
                return cc
            return lax.fori_loop(0, F // 16, zcol, carry)
        lax.fori_loop(0, ZR, zrow, 0)

        rpt = NP // 16
        def zslab(i, carry):
            pltpu.sync_copy(zbuf, acc.at[pl.ds(s * rpt + i * ZR, ZR)])
            return carry
        lax.fori_loop(0, rpt // ZR, zslab, 0)
        plsc.subcore_barrier()

        base = (s * 2 + c) * EPW
        def chunk(ic, carry):
            off = base + ic * CE
            pltpu.sync_copy(src_hbm.at[pl.ds(off, CE)], sidx)
            pltpu.sync_copy(dst_hbm.at[pl.ds(off, CE)], didx)
            pltpu.async_copy(node_hbm.at[sidx], rows, sem).wait()
            pltpu.sync_copy(rows, acc.at[didx], add=True)
            return carry
        lax.fori_loop(0, NCH, chunk, 0)
        plsc.subcore_barrier()

        pltpu.sync_copy(acc.at[pl.ds(s * rpt, rpt)],
                        out_hbm.at[c, pl.ds(s * rpt, rpt)])

    return k(node_pad, src, dst)


def _bn(x, g, b):
    m = jnp.mean(x, axis=0)
    v = jnp.var(x, axis=0)
    return (x - m) / jnp.sqrt(v + 1e-5) * g + b


def _topk_pool(x, w, n, ratio):
    score = x @ w / jnp.linalg.norm(w)
    k = int(math.ceil(ratio * n))
    _, perm = jax.lax.top_k(score, k)
    xp = x[perm] * jnp.tanh(score[perm])[:, None]
    new_id = jnp.full((n,), -1, dtype=jnp.int32).at[perm].set(jnp.arange(k, dtype=jnp.int32))
    return xp, new_id, k


def kernel(node, edge_index, batch, W1_rel, b1_rel, W1_root, bn1_g, bn1_b, pool1_w, W2_rel, b2_rel, W2_root, bn2_g, bn2_b, pool2_w, Wfc, bfc, Wfc1, bfc1):
    src = edge_index[0]
    dst = edge_index[1]
    node_pad = jnp.pad(node, ((0, NP - N), (0, 0)))

    parts = _seg_sum_128_sc(node_pad, src, dst)
    agg = (parts[0] + parts[1])[:N]

    x = agg @ W1_rel + b1_rel + node @ W1_root
    x = _bn(x, bn1_g, bn1_b)
    x = jax.nn.relu(x)
    x, new_id, k1 = _topk_pool(x, pool1_w, N, RATIO)
    src1 = new_id[src]
    dst1 = new_id[dst]
    keep = (src1 >= 0) & (dst1 >= 0)
    x1 = jnp.mean(x, axis=0, keepdims=True)
    msg = x[jnp.where(keep, src1, 0)] * keep[:, None].astype(x.dtype)
    agg2 = jax.ops.segment_sum(msg, jnp.where(keep, dst1, 0), num_segments=k1)
    x = agg2 @ W2_rel + b2_rel + x @ W2_root
    x = _bn(x, bn2_g, bn2_b)
    x = jax.nn.relu(x)
    x, _, k2 = _topk_pool(x, pool2_w, k1, RATIO)
    x2 = jnp.mean(x, axis=0, keepdims=True)
    x = x1 + x2
    x = jax.nn.relu(x @ Wfc + bfc)
    out = x @ Wfc1 + bfc1
    return out


# speedup vs baseline: 12.5840x; 11.6009x over previous
import functools

import jax
import jax.numpy as jnp
import numpy as np
from jax import lax
from jax.experimental import pallas as pl
from jax.experimental.pallas import tpu as pltpu
from jax.experimental.pallas import tpu_sc as plsc

N = 10000
NP = 10240
E = 160000
F = 128
H = 1024
K1 = 5000
K2 = 2500
R = 2048
NBLK = NP // R
RC = 1024
RA = 512
RD = 1024
NEG = np.float32(-3.0e38)
MININT = np.int32(-2147483648)

CE1 = 200
ZR1 = 160


def _seg_sum_128_sc(node_pad, src, dst):
    mesh = plsc.VectorSubcoreMesh(core_axis_name="c", subcore_axis_name="s")
    epw = E // 32

    @functools.partial(
        pl.kernel, mesh=mesh,
        out_type=jax.ShapeDtypeStruct((2, NP, F), jnp.float32),
        scratch_types=[
            pltpu.VMEM((CE1,), jnp.int32),
            pltpu.VMEM((CE1,), jnp.int32),
            pltpu.VMEM((CE1, F), jnp.float32),
            pltpu.VMEM((ZR1, F), jnp.float32),
            pltpu.VMEM_SHARED((NP, F), jnp.float32),
            pltpu.SemaphoreType.DMA,
        ],
        compiler_params=pltpu.CompilerParams(needs_layout_passes=False),
    )
    def k(node_hbm, src_hbm, dst_hbm, out_hbm, sidx, didx, rows, zbuf, acc, sem):
        c = lax.axis_index("c")
        s = lax.axis_index("s")

        def zrow(i, carry):
            def zcol(j, cc):
                zbuf[i, pl.ds(j * 16, 16)] = jnp.zeros((16,), jnp.float32)
                return cc
            return lax.fori_loop(0, F // 16, zcol, carry)
        lax.fori_loop(0, ZR1, zrow, 0)

        rpt = NP // 16
        def zslab(i, carry):
            pltpu.sync_copy(zbuf, acc.at[pl.ds(s * rpt + i * ZR1, ZR1)])
            return carry
        lax.fori_loop(0, rpt // ZR1, zslab, 0)
        plsc.subcore_barrier()

        base = (s * 2 + c) * epw
        def chunk(ic, carry):
            off = base + ic * CE1
            pltpu.sync_copy(src_hbm.at[pl.ds(off, CE1)], sidx)
            pltpu.sync_copy(dst_hbm.at[pl.ds(off, CE1)], didx)
            pltpu.async_copy(node_hbm.at[sidx], rows, sem).wait()
            pltpu.sync_copy(rows, acc.at[didx], add=True)
            return carry
        lax.fori_loop(0, epw // CE1, chunk, 0)
        plsc.subcore_barrier()

        pltpu.sync_copy(acc.at[pl.ds(s * rpt, rpt)],
                        out_hbm.at[c, pl.ds(s * rpt, rpt)])

    return k(node_pad, src, dst)


CE3 = 128
EPT = E // 16
NVR = EPT // 16
HALF = NP // 2
ZR3 = 160


def _seg_sum_1024_sc(xp_flat, src, dst, mask):
    mesh = plsc.VectorSubcoreMesh(core_axis_name="c", subcore_axis_name="s")

    @functools.partial(
        pl.kernel, mesh=mesh,
        out_type=[jax.ShapeDtypeStruct((8 * NP, F), jnp.float32),
                  jax.ShapeDtypeStruct((16, 2, NP), jnp.int32),
                  jax.ShapeDtypeStruct((16, 2, NP), jnp.int32)],
        scratch_types=[
            pltpu.VMEM((NP,), jnp.float32),
            pltpu.VMEM((EPT,), jnp.int32),
            pltpu.VMEM((EPT,), jnp.int32),
            pltpu.VMEM((NP,), jnp.int32),
            pltpu.VMEM((NP,), jnp.int32),
            pltpu.VMEM((CE3,), jnp.int32),
            pltpu.VMEM((CE3,), jnp.int32),
            pltpu.VMEM((CE3, F), jnp.float32),
            pltpu.VMEM((ZR3, F), jnp.float32),
            pltpu.VMEM_SHARED((HALF, F), jnp.float32),
            pltpu.SemaphoreType.DMA,
        ],
        compiler_params=pltpu.CompilerParams(needs_layout_passes=False),
    )
    def k(xp_hbm, src_hbm, dst_hbm, mask_hbm, out_hbm, srcs_hbm, dsts_hbm,
          mask_v, srcb, dstb, src_c, dst_c, sidx, didx, rows, zbuf, acc, sem):
        c = lax.axis_index("c")
        s = lax.axis_index("s")

        pltpu.sync_copy(mask_hbm, mask_v)
        pltpu.sync_copy(src_hbm.at[pl.ds(s * EPT, EPT)], srcb)
        pltpu.sync_copy(dst_hbm.at[pl.ds(s * EPT, EPT)], dstb)

        iota16 = lax.iota(jnp.int32, 16)

        nchs = []
        for h in range(2):
            def prefill(j, carry):
                pad = (iota16 + j * 16) & 127
                src_c[pl.ds(j * 16, 16)] = N + pad
                dst_c[pl.ds(j * 16, 16)] = pad
                return carry
            lax.fori_loop(0, NP // 16, prefill, 0)

            lo = h * HALF
            def compact(j, cnt):
                sv = srcb[pl.ds(j * 16, 16)]
                dv = dstb[pl.ds(j * 16, 16)]
                ms = plsc.load_gather(mask_v, [sv])
                md = plsc.load_gather(mask_v, [dv])
                dvr = dv - lo
                keep = ((ms != 0.0) & (md != 0.0)
                        & (dvr >= 0) & (dvr < HALF))
                plsc.store_compressed(src_c.at[pl.ds(cnt, 16)], sv, mask=keep)
                plsc.store_compressed(dst_c.at[pl.ds(cnt, 16)], dvr, mask=keep)
                pc = plsc.all_reduce_population_count(keep)
                return cnt + jnp.max(pc)
            cnt = lax.fori_loop(0, NVR, compact, jnp.int32(0))
            nchs.append((cnt + CE3 - 1) >> 7)
            pltpu.sync_copy(src_c, srcs_hbm.at[s, h])
            pltpu.sync_copy(dst_c, dsts_hbm.at[s, h])

        def zrow(i, carry):
            def zcol(j, cc):
                zbuf[i, pl.ds(j * 16, 16)] = jnp.zeros((16,), jnp.float32)
                return cc
            return lax.fori_loop(0, F // 16, zcol, carry)
        lax.fori_loop(0, ZR3, zrow, 0)

        rpt = HALF // 16
        for bi in range(4):
            b = c * 4 + bi
            boff = b * NP
            for h in range(2):
                def zslab(i, carry):
                    pltpu.sync_copy(zbuf, acc.at[pl.ds(s * rpt + i * ZR3, ZR3)])
                    return carry
                lax.fori_loop(0, rpt // ZR3, zslab, 0)
                plsc.subcore_barrier()

                def chunk(ic, carry):
                    off = ic * CE3
                    pltpu.sync_copy(srcs_hbm.at[s, h, pl.ds(off, CE3)], sidx)
                    pltpu.sync_copy(dsts_hbm.at[s, h, pl.ds(off, CE3)], didx)
                    def shift(j, cc):
                        sidx[pl.ds(j * 16, 16)] = sidx[pl.ds(j * 16, 16)] + boff
                        return cc
                    lax.fori_loop(0, CE3 // 16, shift, 0)
                    pltpu.async_copy(xp_hbm.at[sidx], rows, sem).wait()
                    pltpu.sync_copy(rows, acc.at[didx], add=True)
                    return carry
                lax.fori_loop(0, nchs[h], chunk, 0)
                plsc.subcore_barrier()

                pltpu.sync_copy(
                    acc.at[pl.ds(s * rpt, rpt)],
                    out_hbm.at[pl.ds(boff + h * HALF + s * rpt, rpt)])
                plsc.subcore_barrier()

    return k(xp_flat, src, dst, mask)[0]


def _sortable(score):
    sb = lax.bitcast_convert_type(score, jnp.int32)
    return jnp.where(sb >= 0, sb, ~sb ^ MININT)


def _threshold(keys, ridx, kk):
    def bit_step(t, p):
        cbit = jnp.int32(1) << (31 - t)
        cand = p | cbit
        cnt = jnp.sum((keys >= (cand ^ MININT)).astype(jnp.int32))
        return jnp.where(cnt >= kk, cand, p)
    p = lax.fori_loop(0, 32, bit_step, jnp.int32(0))
    tau = p ^ MININT
    need = kk - jnp.sum((keys > tau).astype(jnp.int32))

    def tie_step(t, lohi):
        lo, hi = lohi
        mid = (lo + hi) // 2
        cnt = jnp.sum(((keys == tau) & (ridx <= mid)).astype(jnp.int32))
        return jnp.where(cnt >= need, lo, mid + 1), jnp.where(cnt >= need, mid, hi)
    lo, hi = lax.fori_loop(0, 14, tie_step,
                           (jnp.int32(0), jnp.int32(NP - 1)))
    return tau, lo


def _bn_apply(x, stats, g, b, count):
    m = stats[0:1] * (1.0 / count)
    v = stats[1:2] * (1.0 / count) - m * m
    return (x - m) / jnp.sqrt(v + 1e-5) * g + b


def _k2a(parts, node_pad, W1_rel, W1_root, b1):
    def body(parts_ref, node_ref, wr_ref, wo_ref, b1_ref, x_ref, st_ref):
        i = pl.program_id(0)
        agg = parts_ref[0] + parts_ref[1]
        x = (jnp.dot(agg, wr_ref[...], preferred_element_type=jnp.float32)
             + jnp.dot(node_ref[...], wo_ref[...], preferred_element_type=jnp.float32)
             + b1_ref[...])
        rid = i * R + lax.broadcasted_iota(jnp.int32, (R, 1), 0)
        x = jnp.where(rid < N, x, 0.0)
        x_ref[...] = x
        @pl.when(i == 0)
        def _():
            st_ref[...] = jnp.zeros_like(st_ref)
        st_ref[0:1, :] += jnp.sum(x, axis=0, keepdims=True)
        st_ref[1:2, :] += jnp.sum(x * x, axis=0, keepdims=True)

    return pl.pallas_call(
        body,
        grid=(NBLK,),
        in_specs=[
            pl.BlockSpec((2, R, F), lambda i: (0, i, 0)),
            pl.BlockSpec((R, F), lambda i: (i, 0)),
            pl.BlockSpec((F, H), lambda i: (0, 0)),
            pl.BlockSpec((F, H), lambda i: (0, 0)),
            pl.BlockSpec((1, H), lambda i: (0, 0)),
        ],
        out_specs=[
            pl.BlockSpec((R, H), lambda i: (i, 0)),
            pl.BlockSpec((2, H), lambda i: (0, 0)),
        ],
        out_shape=[
            jax.ShapeDtypeStruct((NP, H), jnp.float32),
            jax.ShapeDtypeStruct((2, H), jnp.float32),
        ],
    )(parts, node_pad, W1_rel, W1_root, b1)


def _k2b(x, stats, g1, b1b, w1p):
    def body(x_ref, st_ref, g_ref, b_ref, w_ref, s_ref):
        i = pl.program_id(0)
        w = w_ref[...]
        wn = w / jnp.sqrt(jnp.sum(w * w))
        xbn = jnp.maximum(
            _bn_apply(x_ref[...], st_ref[...], g_ref[...], b_ref[...], N), 0.0)
        sc = jnp.sum(xbn * wn, axis=1, keepdims=True)
        rid = i * R + lax.broadcasted_iota(jnp.int32, (R, 1), 0)
        s_ref[...] = jnp.where(rid < N, sc, NEG)

    return pl.pallas_call(
        body,
        grid=(NBLK,),
        in_specs=[
            pl.BlockSpec((R, H), lambda i: (i, 0)),
            pl.BlockSpec((2, H), lambda i: (0, 0)),
            pl.BlockSpec((1, H), lambda i: (0, 0)),
            pl.BlockSpec((1, H), lambda i: (0, 0)),
            pl.BlockSpec((1, H), lambda i: (0, 0)),
        ],
        out_specs=pl.BlockSpec((R, 1), lambda i: (i, 0)),
        out_shape=jax.ShapeDtypeStruct((NP, 1), jnp.float32),
    )(x, stats, g1, b1b, w1p)


def _k2cd(score, x, stats, g1, b1b):
    def body(sc_all_ref, x_ref, st_ref, g_ref, b_ref,
             xp_ref, mask_ref, x1_ref, sm):
        i = pl.program_id(0)
        @pl.when(i == 0)
        def _():
            keys = _sortable(sc_all_ref[...])
            ridx = lax.broadcasted_iota(jnp.int32, (NP, 1), 0)
            tau, tie_i = _threshold(keys, ridx, K1)
            sm[0] = tau
            sm[1] = tie_i

        tau = sm[0]
        tie_i = sm[1]
        sc = sc_all_ref[pl.ds(i * RC, RC), :]
        key = _sortable(sc)
        rid = i * RC + lax.broadcasted_iota(jnp.int32, (RC, 1), 0)
        m = ((key > tau) | ((key == tau) & (rid <= tie_i))).astype(jnp.float32)
        gate = jnp.tanh(sc) * m
        xbn = jnp.maximum(
            _bn_apply(x_ref[...], st_ref[...], g_ref[...], b_ref[...], N), 0.0)
        xp = xbn * gate
        for cb in range(8):
            xp_ref[cb] = xp[:, cb * F:(cb + 1) * F]
        mask_ref[...] = m
        @pl.when(i == 0)
        def _():
            x1_ref[...] = jnp.zeros_like(x1_ref)
        x1_ref[...] += jnp.sum(xp, axis=0, keepdims=True)
        @pl.when(i == NP // RC - 1)
        def _():
            x1_ref[...] = x1_ref[...] * (1.0 / K1)

    return pl.pallas_call(
        body,
        grid=(NP // RC,),
        in_specs=[
            pl.BlockSpec((NP, 1), lambda i: (0, 0)),
            pl.BlockSpec((RC, H), lambda i: (i, 0)),
            pl.BlockSpec((2, H), lambda i: (0, 0)),
            pl.BlockSpec((1, H), lambda i: (0, 0)),
            pl.BlockSpec((1, H), lambda i: (0, 0)),
        ],
        out_specs=[
            pl.BlockSpec((8, RC, F), lambda i: (0, i, 0)),
            pl.BlockSpec((RC, 1), lambda i: (i, 0)),
            pl.BlockSpec((1, H), lambda i: (0, 0)),
        ],
        out_shape=[
            jax.ShapeDtypeStruct((8, NP, F), jnp.float32),
            jax.ShapeDtypeStruct((NP, 1), jnp.float32),
            jax.ShapeDtypeStruct((1, H), jnp.float32),
        ],
        scratch_shapes=[pltpu.SMEM((2,), jnp.int32)],
        compiler_params=pltpu.CompilerParams(vmem_limit_bytes=52428800),
    )(score, x, stats, g1, b1b)


def _k4a(agg2, xp, W2_rel, W2_root, b2, mask):
    def body(a_ref, xp_ref, wr_ref, wo_ref, b2_ref, m_ref, y_ref, st_ref):
        i = pl.program_id(0)
        y = jnp.broadcast_to(b2_ref[...], (RA, H))
        for cb in range(8):
            y = y + jnp.dot(a_ref[cb], wr_ref[pl.ds(cb * F, F), :],
                            preferred_element_type=jnp.float32)
            y = y + jnp.dot(xp_ref[cb], wo_ref[pl.ds(cb * F, F), :],
                            preferred_element_type=jnp.float32)
        y_ref[...] = y
        ym = y * m_ref[...]
        @pl.when(i == 0)
        def _():
            st_ref[...] = jnp.zeros_like(st_ref)
        st_ref[0:1, :] += jnp.sum(ym, axis=0, keepdims=True)
        st_ref[1:2, :] += jnp.sum(y * ym, axis=0, keepdims=True)

    return pl.pallas_call(
        body,
        grid=(NP // RA,),
        in_specs=[
            pl.BlockSpec((8, RA, F), lambda i: (0, i, 0)),
            pl.BlockSpec((8, RA, F), lambda i: (0, i, 0)),
            pl.BlockSpec((H, H), lambda i: (0, 0)),
            pl.BlockSpec((H, H), lambda i: (0, 0)),
            pl.BlockSpec((1, H), lambda i: (0, 0)),
            pl.BlockSpec((RA, 1), lambda i: (i, 0)),
        ],
        out_specs=[
            pl.BlockSpec((RA, H), lambda i: (i, 0)),
            pl.BlockSpec((2, H), lambda i: (0, 0)),
        ],
        out_shape=[
            jax.ShapeDtypeStruct((NP, H), jnp.float32),
            jax.ShapeDtypeStruct((2, H), jnp.float32),
        ],
        compiler_params=pltpu.CompilerParams(vmem_limit_bytes=52428800),
    )(agg2, xp, W2_rel, W2_root, b2, mask)


def _k4b(y, stats2, g2, b2b, w2p, mask):
    def body(y_ref, st_ref, g_ref, b_ref, w_ref, m_ref, s_ref):
        w = w_ref[...]
        wn = w / jnp.sqrt(jnp.sum(w * w))
        ybn = jnp.maximum(
            _bn_apply(y_ref[...], st_ref[...], g_ref[...], b_ref[...], K1), 0.0)
        sc = jnp.sum(ybn * wn, axis=1, keepdims=True)
        s_ref[...] = jnp.where(m_ref[...] > 0, sc, NEG)

    return pl.pallas_call(
        body,
        grid=(NBLK,),
        in_specs=[
            pl.BlockSpec((R, H), lambda i: (i, 0)),
            pl.BlockSpec((2, H), lambda i: (0, 0)),
            pl.BlockSpec((1, H), lambda i: (0, 0)),
            pl.BlockSpec((1, H), lambda i: (0, 0)),
            pl.BlockSpec((1, H), lambda i: (0, 0)),
            pl.BlockSpec((R, 1), lambda i: (i, 0)),
        ],
        out_specs=pl.BlockSpec((R, 1), lambda i: (i, 0)),
        out_shape=jax.ShapeDtypeStruct((NP, 1), jnp.float32),
    )(y, stats2, g2, b2b, w2p, mask)


def _k4cd(score2, y, stats2, g2, b2b, x1, Wfc, bfc, Wfc1, bfc1):
    def body(sc_all_ref, y_ref, st_ref, g_ref, b_ref, x1_ref,
             wfc_ref, bfc_ref, wfc1_ref, bfc1_ref, out_ref, x2acc, sm):
        i = pl.program_id(0)
        @pl.when(i == 0)
        def _():
            keys = _sortable(sc_all_ref[...])
            ridx = lax.broadcasted_iota(jnp.int32, (NP, 1), 0)
            tau, tie_i = _threshold(keys, ridx, K2)
            sm[0] = tau
            sm[1] = tie_i
            x2acc[...] = jnp.zeros_like(x2acc)

        tau = sm[0]
        tie_i = sm[1]
        sc = sc_all_ref[pl.ds(i * RD, RD), :]
        key = _sortable(sc)
        rid = i * RD + lax.broadcasted_iota(jnp.int32, (RD, 1), 0)
        m2 = ((key > tau) | ((key == tau) & (rid <= tie_i))).astype(jnp.float32)
        gate2 = jnp.tanh(sc) * m2
        ybn = jnp.maximum(
            _bn_apply(y_ref[...], st_ref[...], g_ref[...], b_ref[...], K1), 0.0)
        x2acc[...] += jnp.sum(ybn * gate2, axis=0, keepdims=True)

        @pl.when(i == NP // RD - 1)
        def _():
            xf = x1_ref[...] + x2acc[...] * (1.0 / K2)
            h = jnp.maximum(
                jnp.dot(xf, wfc_ref[...], preferred_element_type=jnp.float32)
                + bfc_ref[...], 0.0)
            out_ref[...] = (jnp.dot(h, wfc1_ref[...],
                                    preferred_element_type=jnp.float32)
                            + bfc1_ref[...])

    return pl.pallas_call(
        body,
        grid=(NP // RD,),
        in_specs=[
            pl.BlockSpec((NP, 1), lambda i: (0, 0)),
            pl.BlockSpec((RD, H), lambda i: (i, 0)),
            pl.BlockSpec((2, H), lambda i: (0, 0)),
            pl.BlockSpec((1, H), lambda i: (0, 0)),
            pl.BlockSpec((1, H), lambda i: (0, 0)),
            pl.BlockSpec((1, H), lambda i: (0, 0)),
            pl.BlockSpec((H, 512), lambda i: (0, 0)),
            pl.BlockSpec((1, 512), lambda i: (0, 0)),
            pl.BlockSpec((512, 5), lambda i: (0, 0)),
            pl.BlockSpec((1, 5), lambda i: (0, 0)),
        ],
        out_specs=pl.BlockSpec((1, 5), lambda i: (0, 0)),
        out_shape=jax.ShapeDtypeStruct((1, 5), jnp.float32),
        scratch_shapes=[pltpu.VMEM((1, H), jnp.float32),
                        pltpu.SMEM((2,), jnp.int32)],
        compiler_params=pltpu.CompilerParams(vmem_limit_bytes=52428800),
    )(score2, y, stats2, g2, b2b, x1, Wfc, bfc, Wfc1, bfc1)


def kernel(node, edge_index, batch, W1_rel, b1_rel, W1_root, bn1_g, bn1_b, pool1_w, W2_rel, b2_rel, W2_root, bn2_g, bn2_b, pool2_w, Wfc, bfc, Wfc1, bfc1):
    src = edge_index[0]
    dst = edge_index[1]
    node_pad = jnp.pad(node, ((0, NP - N), (0, 0)))

    parts = _seg_sum_128_sc(node_pad, src, dst)

    b1 = b1_rel.reshape(1, H)
    x, stats = _k2a(parts, node_pad, W1_rel, W1_root, b1)
    score = _k2b(x, stats, bn1_g.reshape(1, H), bn1_b.reshape(1, H),
                 pool1_w.reshape(1, H))
    xp, mask, x1 = _k2cd(score, x, stats, bn1_g.reshape(1, H),
                         bn1_b.reshape(1, H))

    agg2_flat = _seg_sum_1024_sc(xp.reshape(8 * NP, F), src, dst,
                                 mask.reshape(NP))
    agg2 = agg2_flat.reshape(8, NP, F)

    y, stats2 = _k4a(agg2, xp, W2_rel, W2_root, b2_rel.reshape(1, H), mask)
    score2 = _k4b(y, stats2, bn2_g.reshape(1, H), bn2_b.reshape(1, H),
                  pool2_w.reshape(1, H), mask)
    out = _k4cd(score2, y, stats2, bn2_g.reshape(1, H), bn2_b.reshape(1, H),
                x1, Wfc, bfc.reshape(1, 512), Wfc1, bfc1.reshape(1, 5))
    return out


# speedup vs baseline: 13.4019x; 1.0650x over previous
import functools

import jax
import jax.numpy as jnp
import numpy as np
from jax import lax
from jax.experimental import pallas as pl
from jax.experimental.pallas import tpu as pltpu
from jax.experimental.pallas import tpu_sc as plsc

N = 10000
NP = 10240
E = 160000
F = 128
H = 1024
K1 = 5000
K2 = 2500
R = 2048
NBLK = NP // R
RC = 1024
RA = 512
RD = 1024
NEG = np.float32(-3.0e38)
MININT = np.int32(-2147483648)

EP = 163840
CE1 = 128


def _seg_sum_128_sc(node_pad, src, dst):
    mesh = plsc.VectorSubcoreMesh(core_axis_name="c", subcore_axis_name="s")
    epw = EP // 32

    @functools.partial(
        pl.kernel, mesh=mesh,
        out_type=jax.ShapeDtypeStruct((2, NP, F), jnp.float32),
        scratch_types=[
            pltpu.VMEM((CE1,), jnp.int32),
            pltpu.VMEM((CE1,), jnp.int32),
            pltpu.VMEM((CE1,), jnp.int32),
            pltpu.VMEM((CE1,), jnp.int32),
            pltpu.VMEM((CE1, F), jnp.float32),
            pltpu.VMEM((CE1, F), jnp.float32),
            pltpu.VMEM_SHARED((NP, F), jnp.float32),
            pltpu.SemaphoreType.DMA,
            pltpu.SemaphoreType.DMA,
        ],
        compiler_params=pltpu.CompilerParams(needs_layout_passes=False),
    )
    def k(node_hbm, src_hbm, dst_hbm, out_hbm, sidx, didx, sidx2, didx2,
          rows, rows2, acc, sem, sem2):
        c = lax.axis_index("c")
        s = lax.axis_index("s")

        def zrow(i, carry):
            def zcol(j, cc):
                rows[i, pl.ds(j * 16, 16)] = jnp.zeros((16,), jnp.float32)
                return cc
            return lax.fori_loop(0, F // 16, zcol, carry)
        lax.fori_loop(0, CE1, zrow, 0)

        rpt = NP // 16
        def zslab(i, carry):
            pltpu.sync_copy(rows, acc.at[pl.ds(s * rpt + i * CE1, CE1)])
            return carry
        lax.fori_loop(0, rpt // CE1, zslab, 0)
        plsc.subcore_barrier()

        base = (s * 2 + c) * epw

        def load_start(ic, sx, dx, rw, sm):
            off = base + ic * CE1
            pltpu.sync_copy(src_hbm.at[pl.ds(off, CE1)], sx)
            pltpu.sync_copy(dst_hbm.at[pl.ds(off, CE1)], dx)
            return pltpu.async_copy(node_hbm.at[sx], rw, sm)

        def pair(ip, carry):
            ca = load_start(ip * 2, sidx, didx, rows, sem)
            cb = load_start(ip * 2 + 1, sidx2, didx2, rows2, sem2)
            ca.wait()
            pltpu.sync_copy(rows, acc.at[didx], add=True)
            cb.wait()
            pltpu.sync_copy(rows2, acc.at[didx2], add=True)
            return carry
        lax.fori_loop(0, epw // (2 * CE1), pair, 0)
        plsc.subcore_barrier()

        pltpu.sync_copy(acc.at[pl.ds(s * rpt, rpt)],
                        out_hbm.at[c, pl.ds(s * rpt, rpt)])

    return k(node_pad, src, dst)


CE3 = 128
EPT = EP // 16
NVR = EPT // 16
HALF = NP // 2
FB = 128
ZR3 = 16
CSZ = 1024


def _seg_sum_1024_sc(xp_flat, src, dst, mask):
    mesh = plsc.VectorSubcoreMesh(core_axis_name="c", subcore_axis_name="s")

    @functools.partial(
        pl.kernel, mesh=mesh,
        out_type=[jax.ShapeDtypeStruct((8 * NP, FB), jnp.float32),
                  jax.ShapeDtypeStruct((16, 2, NP), jnp.int32),
                  jax.ShapeDtypeStruct((16, 2, NP), jnp.int32)],
        scratch_types=[
            pltpu.VMEM((NP,), jnp.float32),
            pltpu.VMEM((CSZ,), jnp.int32),
            pltpu.VMEM((CSZ,), jnp.int32),
            pltpu.VMEM((NP,), jnp.int32),
            pltpu.VMEM((NP,), jnp.int32),
            pltpu.VMEM((CE3,), jnp.int32),
            pltpu.VMEM((CE3,), jnp.int32),
            pltpu.VMEM((CE3,), jnp.int32),
            pltpu.VMEM((CE3,), jnp.int32),
            pltpu.VMEM((CE3, FB), jnp.float32),
            pltpu.VMEM((CE3, FB), jnp.float32),
            pltpu.VMEM((ZR3, FB), jnp.float32),
            pltpu.VMEM_SHARED((HALF, FB), jnp.float32),
            pltpu.SemaphoreType.DMA,
            pltpu.SemaphoreType.DMA,
        ],
        compiler_params=pltpu.CompilerParams(needs_layout_passes=False),
    )
    def k(xp_hbm, src_hbm, dst_hbm, mask_hbm, out_hbm, srcs_hbm, dsts_hbm,
          mask_v, srcb, dstb, src_c, dst_c, sidx, didx, sidx2, didx2,
          rows, rows2, zbuf, acc, sem, sem2):
        c = lax.axis_index("c")
        s = lax.axis_index("s")

        pltpu.sync_copy(mask_hbm, mask_v)

        iota16 = lax.iota(jnp.int32, 16)

        nchs = []
        for h in range(2):
            def prefill(j, carry):
                pad = (iota16 + j * 16) & 127
                src_c[pl.ds(j * 16, 16)] = N + pad
                dst_c[pl.ds(j * 16, 16)] = pad
                return carry
            lax.fori_loop(0, NP // 16, prefill, 0)

            lo = h * HALF
            def outer(oc, cnt0):
                pltpu.sync_copy(src_hbm.at[pl.ds(s * EPT + oc * CSZ, CSZ)], srcb)
                pltpu.sync_copy(dst_hbm.at[pl.ds(s * EPT + oc * CSZ, CSZ)], dstb)
                def compact(j, cnt):
                    sv = srcb[pl.ds(j * 16, 16)]
                    dv = dstb[pl.ds(j * 16, 16)]
                    ms = plsc.load_gather(mask_v, [sv])
                    md = plsc.load_gather(mask_v, [dv])
                    dvr = dv - lo
                    keep = ((ms != 0.0) & (md != 0.0)
                            & (dvr >= 0) & (dvr < HALF))
                    plsc.store_compressed(src_c.at[pl.ds(cnt, 16)], sv, mask=keep)
                    plsc.store_compressed(dst_c.at[pl.ds(cnt, 16)], dvr, mask=keep)
                    pc = plsc.all_reduce_population_count(keep)
                    return cnt + jnp.max(pc)
                return lax.fori_loop(0, CSZ // 16, compact, cnt0)
            cnt = lax.fori_loop(0, EPT // CSZ, outer, jnp.int32(0))
            nchs.append((cnt + 2 * CE3 - 1) >> 8)
            pltpu.sync_copy(src_c, srcs_hbm.at[s, h])
            pltpu.sync_copy(dst_c, dsts_hbm.at[s, h])

        def zrow(i, carry):
            def zcol(j, cc):
                zbuf[i, pl.ds(j * 16, 16)] = jnp.zeros((16,), jnp.float32)
                return cc
            return lax.fori_loop(0, FB // 16, zcol, carry)
        lax.fori_loop(0, ZR3, zrow, 0)

        rpt = HALF // 16
        for si in range(4):
            b = c * 4 + si
            boff = b * NP
            for h in range(2):
                def zslab(i, carry):
                    pltpu.sync_copy(zbuf, acc.at[pl.ds(s * rpt + i * ZR3, ZR3)])
                    return carry
                lax.fori_loop(0, rpt // ZR3, zslab, 0)
                plsc.subcore_barrier()

                def load_start(ic, sx, dx, rw, sm):
                    off = ic * CE3
                    pltpu.sync_copy(srcs_hbm.at[s, h, pl.ds(off, CE3)], sx)
                    pltpu.sync_copy(dsts_hbm.at[s, h, pl.ds(off, CE3)], dx)
                    def shift(j, cc):
                        sx[pl.ds(j * 16, 16)] = sx[pl.ds(j * 16, 16)] + boff
                        return cc
                    lax.fori_loop(0, CE3 // 16, shift, 0)
                    return pltpu.async_copy(xp_hbm.at[sx], rw, sm)

                def pair(ip, carry):
                    ca = load_start(ip * 2, sidx, didx, rows, sem)
                    cb = load_start(ip * 2 + 1, sidx2, didx2, rows2, sem2)
                    ca.wait()
                    pltpu.sync_copy(rows, acc.at[didx], add=True)
                    cb.wait()
                    pltpu.sync_copy(rows2, acc.at[didx2], add=True)
                    return carry
                lax.fori_loop(0, nchs[h], pair, 0)
                plsc.subcore_barrier()

                pltpu.sync_copy(
                    acc.at[pl.ds(s * rpt, rpt)],
                    out_hbm.at[pl.ds(boff + h * HALF + s * rpt, rpt)])
                plsc.subcore_barrier()

    return k(xp_flat, src, dst, mask)[0]


def _sortable(score):
    sb = lax.bitcast_convert_type(score, jnp.int32)
    return jnp.where(sb >= 0, sb, ~sb ^ MININT)


def _threshold(keys, ridx, kk):
    def bit_step(t, p):
        cbit = jnp.int32(1) << (31 - t)
        cand = p | cbit
        cnt = jnp.sum((keys >= (cand ^ MININT)).astype(jnp.int32))
        return jnp.where(cnt >= kk, cand, p)
    p = lax.fori_loop(0, 32, bit_step, jnp.int32(0))
    tau = p ^ MININT
    need = kk - jnp.sum((keys > tau).astype(jnp.int32))

    def tie_step(t, lohi):
        lo, hi = lohi
        mid = (lo + hi) // 2
        cnt = jnp.sum(((keys == tau) & (ridx <= mid)).astype(jnp.int32))
        return jnp.where(cnt >= need, lo, mid + 1), jnp.where(cnt >= need, mid, hi)
    lo, hi = lax.fori_loop(0, 14, tie_step,
                           (jnp.int32(0), jnp.int32(NP - 1)))
    return tau, lo


def _bn_apply(x, stats, g, b, count):
    m = stats[0:1] * (1.0 / count)
    v = stats[1:2] * (1.0 / count) - m * m
    return (x - m) / jnp.sqrt(v + 1e-5) * g + b


def _k2a(parts, node_pad, W1_rel, W1_root, b1):
    def body(parts_ref, node_ref, wr_ref, wo_ref, b1_ref, x_ref, st_ref):
        i = pl.program_id(0)
        agg = parts_ref[0] + parts_ref[1]
        x = (jnp.dot(agg, wr_ref[...], preferred_element_type=jnp.float32)
             + jnp.dot(node_ref[...], wo_ref[...], preferred_element_type=jnp.float32)
             + b1_ref[...])
        rid = i * R + lax.broadcasted_iota(jnp.int32, (R, 1), 0)
        x = jnp.where(rid < N, x, 0.0)
        x_ref[...] = x
        @pl.when(i == 0)
        def _():
            st_ref[...] = jnp.zeros_like(st_ref)
        st_ref[0:1, :] += jnp.sum(x, axis=0, keepdims=True)
        st_ref[1:2, :] += jnp.sum(x * x, axis=0, keepdims=True)

    return pl.pallas_call(
        body,
        grid=(NBLK,),
        in_specs=[
            pl.BlockSpec((2, R, F), lambda i: (0, i, 0)),
            pl.BlockSpec((R, F), lambda i: (i, 0)),
            pl.BlockSpec((F, H), lambda i: (0, 0)),
            pl.BlockSpec((F, H), lambda i: (0, 0)),
            pl.BlockSpec((1, H), lambda i: (0, 0)),
        ],
        out_specs=[
            pl.BlockSpec((R, H), lambda i: (i, 0)),
            pl.BlockSpec((2, H), lambda i: (0, 0)),
        ],
        out_shape=[
            jax.ShapeDtypeStruct((NP, H), jnp.float32),
            jax.ShapeDtypeStruct((2, H), jnp.float32),
        ],
    )(parts, node_pad, W1_rel, W1_root, b1)


def _k2b(x, stats, g1, b1b, w1p):
    def body(x_ref, st_ref, g_ref, b_ref, w_ref, s_ref):
        i = pl.program_id(0)
        w = w_ref[...]
        wn = w / jnp.sqrt(jnp.sum(w * w))
        xbn = jnp.maximum(
            _bn_apply(x_ref[...], st_ref[...], g_ref[...], b_ref[...], N), 0.0)
        sc = jnp.sum(xbn * wn, axis=1, keepdims=True)
        rid = i * R + lax.broadcasted_iota(jnp.int32, (R, 1), 0)
        s_ref[...] = jnp.where(rid < N, sc, NEG)

    return pl.pallas_call(
        body,
        grid=(NBLK,),
        in_specs=[
            pl.BlockSpec((R, H), lambda i: (i, 0)),
            pl.BlockSpec((2, H), lambda i: (0, 0)),
            pl.BlockSpec((1, H), lambda i: (0, 0)),
            pl.BlockSpec((1, H), lambda i: (0, 0)),
            pl.BlockSpec((1, H), lambda i: (0, 0)),
        ],
        out_specs=pl.BlockSpec((R, 1), lambda i: (i, 0)),
        out_shape=jax.ShapeDtypeStruct((NP, 1), jnp.float32),
    )(x, stats, g1, b1b, w1p)


def _k2cd(score, x, stats, g1, b1b):
    def body(sc_all_ref, x_ref, st_ref, g_ref, b_ref,
             xp_ref, mask_ref, x1_ref, sm):
        i = pl.program_id(0)
        @pl.when(i == 0)
        def _():
            keys = _sortable(sc_all_ref[...])
            ridx = lax.broadcasted_iota(jnp.int32, (NP, 1), 0)
            tau, tie_i = _threshold(keys, ridx, K1)
            sm[0] = tau
            sm[1] = tie_i

        tau = sm[0]
        tie_i = sm[1]
        sc = sc_all_ref[pl.ds(i * RC, RC), :]
        key = _sortable(sc)
        rid = i * RC + lax.broadcasted_iota(jnp.int32, (RC, 1), 0)
        m = ((key > tau) | ((key == tau) & (rid <= tie_i))).astype(jnp.float32)
        gate = jnp.tanh(sc) * m
        xbn = jnp.maximum(
            _bn_apply(x_ref[...], st_ref[...], g_ref[...], b_ref[...], N), 0.0)
        xp = xbn * gate
        for cb in range(8):
            xp_ref[cb] = xp[:, cb * FB:(cb + 1) * FB]
        mask_ref[...] = m
        @pl.when(i == 0)
        def _():
            x1_ref[...] = jnp.zeros_like(x1_ref)
        x1_ref[...] += jnp.sum(xp, axis=0, keepdims=True)
        @pl.when(i == NP // RC - 1)
        def _():
            x1_ref[...] = x1_ref[...] * (1.0 / K1)

    return pl.pallas_call(
        body,
        grid=(NP // RC,),
        in_specs=[
            pl.BlockSpec((NP, 1), lambda i: (0, 0)),
            pl.BlockSpec((RC, H), lambda i: (i, 0)),
            pl.BlockSpec((2, H), lambda i: (0, 0)),
            pl.BlockSpec((1, H), lambda i: (0, 0)),
            pl.BlockSpec((1, H), lambda i: (0, 0)),
        ],
        out_specs=[
            pl.BlockSpec((8, RC, FB), lambda i: (0, i, 0)),
            pl.BlockSpec((RC, 1), lambda i: (i, 0)),
            pl.BlockSpec((1, H), lambda i: (0, 0)),
        ],
        out_shape=[
            jax.ShapeDtypeStruct((8, NP, FB), jnp.float32),
            jax.ShapeDtypeStruct((NP, 1), jnp.float32),
            jax.ShapeDtypeStruct((1, H), jnp.float32),
        ],
        scratch_shapes=[pltpu.SMEM((2,), jnp.int32)],
        compiler_params=pltpu.CompilerParams(vmem_limit_bytes=52428800),
    )(score, x, stats, g1, b1b)


def _k4a(agg2, xp, W2_rel, W2_root, b2, mask):
    def body(a_ref, xp_ref, wr_ref, wo_ref, b2_ref, m_ref, y_ref, st_ref):
        i = pl.program_id(0)
        y = jnp.broadcast_to(b2_ref[...], (RA, H))
        for cb in range(8):
            y = y + jnp.dot(a_ref[cb], wr_ref[pl.ds(cb * FB, FB), :],
                            preferred_element_type=jnp.float32)
            y = y + jnp.dot(xp_ref[cb], wo_ref[pl.ds(cb * FB, FB), :],
                            preferred_element_type=jnp.float32)
        y_ref[...] = y
        ym = y * m_ref[...]
        @pl.when(i == 0)
        def _():
            st_ref[...] = jnp.zeros_like(st_ref)
        st_ref[0:1, :] += jnp.sum(ym, axis=0, keepdims=True)
        st_ref[1:2, :] += jnp.sum(y * ym, axis=0, keepdims=True)

    return pl.pallas_call(
        body,
        grid=(NP // RA,),
        in_specs=[
            pl.BlockSpec((8, RA, FB), lambda i: (0, i, 0)),
            pl.BlockSpec((8, RA, FB), lambda i: (0, i, 0)),
            pl.BlockSpec((H, H), lambda i: (0, 0)),
            pl.BlockSpec((H, H), lambda i: (0, 0)),
            pl.BlockSpec((1, H), lambda i: (0, 0)),
            pl.BlockSpec((RA, 1), lambda i: (i, 0)),
        ],
        out_specs=[
            pl.BlockSpec((RA, H), lambda i: (i, 0)),
            pl.BlockSpec((2, H), lambda i: (0, 0)),
        ],
        out_shape=[
            jax.ShapeDtypeStruct((NP, H), jnp.float32),
            jax.ShapeDtypeStruct((2, H), jnp.float32),
        ],
        compiler_params=pltpu.CompilerParams(vmem_limit_bytes=52428800),
    )(agg2, xp, W2_rel, W2_root, b2, mask)


def _k4b(y, stats2, g2, b2b, w2p, mask):
    def body(y_ref, st_ref, g_ref, b_ref, w_ref, m_ref, s_ref):
        w = w_ref[...]
        wn = w / jnp.sqrt(jnp.sum(w * w))
        ybn = jnp.maximum(
            _bn_apply(y_ref[...], st_ref[...], g_ref[...], b_ref[...], K1), 0.0)
        sc = jnp.sum(ybn * wn, axis=1, keepdims=True)
        s_ref[...] = jnp.where(m_ref[...] > 0, sc, NEG)

    return pl.pallas_call(
        body,
        grid=(NBLK,),
        in_specs=[
            pl.BlockSpec((R, H), lambda i: (i, 0)),
            pl.BlockSpec((2, H), lambda i: (0, 0)),
            pl.BlockSpec((1, H), lambda i: (0, 0)),
            pl.BlockSpec((1, H), lambda i: (0, 0)),
            pl.BlockSpec((1, H), lambda i: (0, 0)),
            pl.BlockSpec((R, 1), lambda i: (i, 0)),
        ],
        out_specs=pl.BlockSpec((R, 1), lambda i: (i, 0)),
        out_shape=jax.ShapeDtypeStruct((NP, 1), jnp.float32),
    )(y, stats2, g2, b2b, w2p, mask)


def _k4cd(score2, y, stats2, g2, b2b, x1, Wfc, bfc, Wfc1, bfc1):
    def body(sc_all_ref, y_ref, st_ref, g_ref, b_ref, x1_ref,
             wfc_ref, bfc_ref, wfc1_ref, bfc1_ref, out_ref, x2acc, sm):
        i = pl.program_id(0)
        @pl.when(i == 0)
        def _():
            keys = _sortable(sc_all_ref[...])
            ridx = lax.broadcasted_iota(jnp.int32, (NP, 1), 0)
            tau, tie_i = _threshold(keys, ridx, K2)
            sm[0] = tau
            sm[1] = tie_i
            x2acc[...] = jnp.zeros_like(x2acc)

        tau = sm[0]
        tie_i = sm[1]
        sc = sc_all_ref[pl.ds(i * RD, RD), :]
        key = _sortable(sc)
        rid = i * RD + lax.broadcasted_iota(jnp.int32, (RD, 1), 0)
        m2 = ((key > tau) | ((key == tau) & (rid <= tie_i))).astype(jnp.float32)
        gate2 = jnp.tanh(sc) * m2
        ybn = jnp.maximum(
            _bn_apply(y_ref[...], st_ref[...], g_ref[...], b_ref[...], K1), 0.0)
        x2acc[...] += jnp.sum(ybn * gate2, axis=0, keepdims=True)

        @pl.when(i == NP // RD - 1)
        def _():
            xf = x1_ref[...] + x2acc[...] * (1.0 / K2)
            h = jnp.maximum(
                jnp.dot(xf, wfc_ref[...], preferred_element_type=jnp.float32)
                + bfc_ref[...], 0.0)
            out_ref[...] = (jnp.dot(h, wfc1_ref[...],
                                    preferred_element_type=jnp.float32)
                            + bfc1_ref[...])

    return pl.pallas_call(
        body,
        grid=(NP // RD,),
        in_specs=[
            pl.BlockSpec((NP, 1), lambda i: (0, 0)),
            pl.BlockSpec((RD, H), lambda i: (i, 0)),
            pl.BlockSpec((2, H), lambda i: (0, 0)),
            pl.BlockSpec((1, H), lambda i: (0, 0)),
            pl.BlockSpec((1, H), lambda i: (0, 0)),
            pl.BlockSpec((1, H), lambda i: (0, 0)),
            pl.BlockSpec((H, 512), lambda i: (0, 0)),
            pl.BlockSpec((1, 512), lambda i: (0, 0)),
            pl.BlockSpec((512, 5), lambda i: (0, 0)),
            pl.BlockSpec((1, 5), lambda i: (0, 0)),
        ],
        out_specs=pl.BlockSpec((1, 5), lambda i: (0, 0)),
        out_shape=jax.ShapeDtypeStruct((1, 5), jnp.float32),
        scratch_shapes=[pltpu.VMEM((1, H), jnp.float32),
                        pltpu.SMEM((2,), jnp.int32)],
        compiler_params=pltpu.CompilerParams(vmem_limit_bytes=52428800),
    )(score2, y, stats2, g2, b2b, x1, Wfc, bfc, Wfc1, bfc1)


def kernel(node, edge_index, batch, W1_rel, b1_rel, W1_root, bn1_g, bn1_b, pool1_w, W2_rel, b2_rel, W2_root, bn2_g, bn2_b, pool2_w, Wfc, bfc, Wfc1, bfc1):
    pad_idx = jnp.asarray(N + (np.arange(EP - E) % 128), dtype=jnp.int32)
    src = jnp.concatenate([edge_index[0], pad_idx])
    dst = jnp.concatenate([edge_index[1], pad_idx])
    node_pad = jnp.pad(node, ((0, NP - N), (0, 0)))

    parts = _seg_sum_128_sc(node_pad, src, dst)

    b1 = b1_rel.reshape(1, H)
    x, stats = _k2a(parts, node_pad, W1_rel, W1_root, b1)
    score = _k2b(x, stats, bn1_g.reshape(1, H), bn1_b.reshape(1, H),
                 pool1_w.reshape(1, H))
    xp, mask, x1 = _k2cd(score, x, stats, bn1_g.reshape(1, H),
                         bn1_b.reshape(1, H))

    agg2_flat = _seg_sum_1024_sc(xp.reshape(8 * NP, FB), src, dst,
                                 mask.reshape(NP))
    agg2 = agg2_flat.reshape(8, NP, FB)

    y, stats2 = _k4a(agg2, xp, W2_rel, W2_root, b2_rel.reshape(1, H), mask)
    score2 = _k4b(y, stats2, bn2_g.reshape(1, H), bn2_b.reshape(1, H),
                  pool2_w.reshape(1, H), mask)
    out = _k4cd(score2, y, stats2, bn2_g.reshape(1, H), bn2_b.reshape(1, H),
                x1, Wfc, bfc.reshape(1, 512), Wfc1, bfc1.reshape(1, 5))
    return out


# speedup vs baseline: 14.1334x; 1.0546x over previous
import functools

import jax
import jax.numpy as jnp
import numpy as np
from jax import lax
from jax.experimental import pallas as pl
from jax.experimental.pallas import tpu as pltpu
from jax.experimental.pallas import tpu_sc as plsc

N = 10000
NP = 10240
E = 160000
F = 128
H = 1024
K1 = 5000
K2 = 2500
R = 2048
NBLK = NP // R
RC = 1024
RA = 512
RD = 1024
NEG = np.float32(-3.0e38)
MININT = np.int32(-2147483648)

EP = 163840
CE1 = 128


def _seg_sum_128_sc(node_pad, src, dst):
    mesh = plsc.VectorSubcoreMesh(core_axis_name="c", subcore_axis_name="s")
    epw = EP // 32

    @functools.partial(
        pl.kernel, mesh=mesh,
        out_type=jax.ShapeDtypeStruct((2, NP, F), jnp.float32),
        scratch_types=[
            pltpu.VMEM((CE1,), jnp.int32),
            pltpu.VMEM((CE1,), jnp.int32),
            pltpu.VMEM((CE1,), jnp.int32),
            pltpu.VMEM((CE1,), jnp.int32),
            pltpu.VMEM((CE1, F), jnp.float32),
            pltpu.VMEM((CE1, F), jnp.float32),
            pltpu.VMEM_SHARED((NP, F), jnp.float32),
            pltpu.SemaphoreType.DMA,
            pltpu.SemaphoreType.DMA,
        ],
        compiler_params=pltpu.CompilerParams(needs_layout_passes=False),
    )
    def k(node_hbm, src_hbm, dst_hbm, out_hbm, sidx, didx, sidx2, didx2,
          rows, rows2, acc, sem, sem2):
        c = lax.axis_index("c")
        s = lax.axis_index("s")

        def zrow(i, carry):
            def zcol(j, cc):
                rows[i, pl.ds(j * 16, 16)] = jnp.zeros((16,), jnp.float32)
                return cc
            return lax.fori_loop(0, F // 16, zcol, carry)
        lax.fori_loop(0, CE1, zrow, 0)

        rpt = NP // 16
        def zslab(i, carry):
            pltpu.sync_copy(rows, acc.at[pl.ds(s * rpt + i * CE1, CE1)])
            return carry
        lax.fori_loop(0, rpt // CE1, zslab, 0)
        plsc.subcore_barrier()

        base = (s * 2 + c) * epw

        def load_start(ic, sx, dx, rw, sm):
            off = base + ic * CE1
            pltpu.sync_copy(src_hbm.at[pl.ds(off, CE1)], sx)
            pltpu.sync_copy(dst_hbm.at[pl.ds(off, CE1)], dx)
            return pltpu.async_copy(node_hbm.at[sx], rw, sm)

        def pair(ip, carry):
            ca = load_start(ip * 2, sidx, didx, rows, sem)
            cb = load_start(ip * 2 + 1, sidx2, didx2, rows2, sem2)
            ca.wait()
            pltpu.sync_copy(rows, acc.at[didx], add=True)
            cb.wait()
            pltpu.sync_copy(rows2, acc.at[didx2], add=True)
            return carry
        lax.fori_loop(0, epw // (2 * CE1), pair, 0)
        plsc.subcore_barrier()

        pltpu.sync_copy(acc.at[pl.ds(s * rpt, rpt)],
                        out_hbm.at[c, pl.ds(s * rpt, rpt)])

    return k(node_pad, src, dst)


CE3 = 128
EPT = EP // 16
NVR = EPT // 16
HALF = NP // 2
FB = 128
ZR3 = 16
CSZ = 1024


def _seg_sum_1024_sc(xp_flat, src, dst, mask):
    mesh = plsc.VectorSubcoreMesh(core_axis_name="c", subcore_axis_name="s")

    @functools.partial(
        pl.kernel, mesh=mesh,
        out_type=[jax.ShapeDtypeStruct((8 * NP, FB), jnp.float32),
                  jax.ShapeDtypeStruct((16, 2, NP), jnp.int32),
                  jax.ShapeDtypeStruct((16, 2, NP), jnp.int32)],
        scratch_types=[
            pltpu.VMEM((NP,), jnp.float32),
            pltpu.VMEM((CSZ,), jnp.int32),
            pltpu.VMEM((CSZ,), jnp.int32),
            pltpu.VMEM((NP,), jnp.int32),
            pltpu.VMEM((NP,), jnp.int32),
            pltpu.VMEM((CE3,), jnp.int32),
            pltpu.VMEM((CE3,), jnp.int32),
            pltpu.VMEM((CE3,), jnp.int32),
            pltpu.VMEM((CE3,), jnp.int32),
            pltpu.VMEM((CE3, FB), jnp.float32),
            pltpu.VMEM((CE3, FB), jnp.float32),
            pltpu.VMEM((ZR3, FB), jnp.float32),
            pltpu.VMEM_SHARED((HALF, FB), jnp.float32),
            pltpu.SemaphoreType.DMA,
            pltpu.SemaphoreType.DMA,
        ],
        compiler_params=pltpu.CompilerParams(needs_layout_passes=False),
    )
    def k(xp_hbm, src_hbm, dst_hbm, mask_hbm, out_hbm, srcs_hbm, dsts_hbm,
          mask_v, srcb, dstb, src_c, dst_c, sidx, didx, sidx2, didx2,
          rows, rows2, zbuf, acc, sem, sem2):
        c = lax.axis_index("c")
        s = lax.axis_index("s")

        pltpu.sync_copy(mask_hbm, mask_v)

        iota16 = lax.iota(jnp.int32, 16)

        nchs = []
        for h in range(2):
            def prefill(j, carry):
                pad = (iota16 + j * 16) & 127
                src_c[pl.ds(j * 16, 16)] = N + pad
                dst_c[pl.ds(j * 16, 16)] = pad
                return carry
            lax.fori_loop(0, NP // 16, prefill, 0)

            lo = h * HALF
            def outer(oc, cnt0):
                pltpu.sync_copy(src_hbm.at[pl.ds(s * EPT + oc * CSZ, CSZ)], srcb)
                pltpu.sync_copy(dst_hbm.at[pl.ds(s * EPT + oc * CSZ, CSZ)], dstb)
                def compact(j, cnt):
                    sv = srcb[pl.ds(j * 16, 16)]
                    dv = dstb[pl.ds(j * 16, 16)]
                    ms = plsc.load_gather(mask_v, [sv])
                    md = plsc.load_gather(mask_v, [dv])
                    dvr = dv - lo
                    keep = ((ms != 0.0) & (md != 0.0)
                            & (dvr >= 0) & (dvr < HALF))
                    plsc.store_compressed(src_c.at[pl.ds(cnt, 16)], sv, mask=keep)
                    plsc.store_compressed(dst_c.at[pl.ds(cnt, 16)], dvr, mask=keep)
                    pc = plsc.all_reduce_population_count(keep)
                    return cnt + jnp.max(pc)
                return lax.fori_loop(0, CSZ // 16, compact, cnt0)
            cnt = lax.fori_loop(0, EPT // CSZ, outer, jnp.int32(0))
            nchs.append((cnt + 2 * CE3 - 1) >> 8)
            pltpu.sync_copy(src_c, srcs_hbm.at[s, h])
            pltpu.sync_copy(dst_c, dsts_hbm.at[s, h])

        def zrow(i, carry):
            def zcol(j, cc):
                zbuf[i, pl.ds(j * 16, 16)] = jnp.zeros((16,), jnp.float32)
                return cc
            return lax.fori_loop(0, FB // 16, zcol, carry)
        lax.fori_loop(0, ZR3, zrow, 0)

        rpt = HALF // 16
        for si in range(4):
            b = c * 4 + si
            boff = b * NP
            for h in range(2):
                def zslab(i, carry):
                    pltpu.sync_copy(zbuf, acc.at[pl.ds(s * rpt + i * ZR3, ZR3)])
                    return carry
                lax.fori_loop(0, rpt // ZR3, zslab, 0)
                plsc.subcore_barrier()

                def load_start(ic, sx, dx, rw, sm):
                    off = ic * CE3
                    pltpu.sync_copy(srcs_hbm.at[s, h, pl.ds(off, CE3)], sx)
                    pltpu.sync_copy(dsts_hbm.at[s, h, pl.ds(off, CE3)], dx)
                    def shift(j, cc):
                        sx[pl.ds(j * 16, 16)] = sx[pl.ds(j * 16, 16)] + boff
                        return cc
                    lax.fori_loop(0, CE3 // 16, shift, 0)
                    return pltpu.async_copy(xp_hbm.at[sx], rw, sm)

                def pair(ip, carry):
                    ca = load_start(ip * 2, sidx, didx, rows, sem)
                    cb = load_start(ip * 2 + 1, sidx2, didx2, rows2, sem2)
                    ca.wait()
                    pltpu.sync_copy(rows, acc.at[didx], add=True)
                    cb.wait()
                    pltpu.sync_copy(rows2, acc.at[didx2], add=True)
                    return carry
                lax.fori_loop(0, nchs[h], pair, 0)
                plsc.subcore_barrier()

                pltpu.sync_copy(
                    acc.at[pl.ds(s * rpt, rpt)],
                    out_hbm.at[pl.ds(boff + h * HALF + s * rpt, rpt)])
                plsc.subcore_barrier()

    return k(xp_flat, src, dst, mask)[0]


def _sortable(score):
    sb = lax.bitcast_convert_type(score, jnp.int32)
    return jnp.where(sb >= 0, sb, ~sb ^ MININT)


def _threshold(keys, ridx, kk):
    def bit_step(t, p):
        cbit = jnp.int32(1) << (31 - t)
        cand = p | cbit
        cnt = jnp.sum((keys >= (cand ^ MININT)).astype(jnp.int32))
        return jnp.where(cnt >= kk, cand, p)
    p = lax.fori_loop(0, 32, bit_step, jnp.int32(0))
    tau = p ^ MININT
    need = kk - jnp.sum((keys > tau).astype(jnp.int32))

    def tie_step(t, lohi):
        lo, hi = lohi
        mid = (lo + hi) // 2
        cnt = jnp.sum(((keys == tau) & (ridx <= mid)).astype(jnp.int32))
        return jnp.where(cnt >= need, lo, mid + 1), jnp.where(cnt >= need, mid, hi)
    lo, hi = lax.fori_loop(0, 14, tie_step,
                           (jnp.int32(0), jnp.int32(NP - 1)))
    return tau, lo


def _bn_apply(x, stats, g, b, count):
    m = stats[0:1] * (1.0 / count)
    v = stats[1:2] * (1.0 / count) - m * m
    return (x - m) / jnp.sqrt(v + 1e-5) * g + b


def _k2a(parts, node_pad, W1_rel, W1_root, b1):
    def body(parts_ref, node_ref, wr_ref, wo_ref, b1_ref, x_ref, st_ref):
        i = pl.program_id(0)
        agg = parts_ref[0] + parts_ref[1]
        x = (jnp.dot(agg, wr_ref[...], preferred_element_type=jnp.float32)
             + jnp.dot(node_ref[...], wo_ref[...], preferred_element_type=jnp.float32)
             + b1_ref[...])
        rid = i * R + lax.broadcasted_iota(jnp.int32, (R, 1), 0)
        x = jnp.where(rid < N, x, 0.0)
        x_ref[...] = x
        @pl.when(i == 0)
        def _():
            st_ref[...] = jnp.zeros_like(st_ref)
        st_ref[0:1, :] += jnp.sum(x, axis=0, keepdims=True)
        st_ref[1:2, :] += jnp.sum(x * x, axis=0, keepdims=True)

    return pl.pallas_call(
        body,
        grid=(NBLK,),
        in_specs=[
            pl.BlockSpec((2, R, F), lambda i: (0, i, 0)),
            pl.BlockSpec((R, F), lambda i: (i, 0)),
            pl.BlockSpec((F, H), lambda i: (0, 0)),
            pl.BlockSpec((F, H), lambda i: (0, 0)),
            pl.BlockSpec((1, H), lambda i: (0, 0)),
        ],
        out_specs=[
            pl.BlockSpec((R, H), lambda i: (i, 0)),
            pl.BlockSpec((2, H), lambda i: (0, 0)),
        ],
        out_shape=[
            jax.ShapeDtypeStruct((NP, H), jnp.float32),
            jax.ShapeDtypeStruct((2, H), jnp.float32),
        ],
    )(parts, node_pad, W1_rel, W1_root, b1)


def _k2b(x, stats, g1, b1b, w1p):
    def body(x_ref, st_ref, g_ref, b_ref, w_ref, s_ref):
        i = pl.program_id(0)
        w = w_ref[...]
        wn = w / jnp.sqrt(jnp.sum(w * w))
        xbn = jnp.maximum(
            _bn_apply(x_ref[...], st_ref[...], g_ref[...], b_ref[...], N), 0.0)
        sc = jnp.sum(xbn * wn, axis=1, keepdims=True)
        rid = i * R + lax.broadcasted_iota(jnp.int32, (R, 1), 0)
        s_ref[...] = jnp.where(rid < N, sc, NEG)

    return pl.pallas_call(
        body,
        grid=(NBLK,),
        in_specs=[
            pl.BlockSpec((R, H), lambda i: (i, 0)),
            pl.BlockSpec((2, H), lambda i: (0, 0)),
            pl.BlockSpec((1, H), lambda i: (0, 0)),
            pl.BlockSpec((1, H), lambda i: (0, 0)),
            pl.BlockSpec((1, H), lambda i: (0, 0)),
        ],
        out_specs=pl.BlockSpec((R, 1), lambda i: (i, 0)),
        out_shape=jax.ShapeDtypeStruct((NP, 1), jnp.float32),
    )(x, stats, g1, b1b, w1p)


def _k2cd(score, x, stats, g1, b1b):
    def body(sc_all_ref, x_ref, st_ref, g_ref, b_ref,
             xp_ref, mask_ref, x1_ref, sm):
        i = pl.program_id(0)
        @pl.when(i == 0)
        def _():
            keys = _sortable(sc_all_ref[...])
            ridx = lax.broadcasted_iota(jnp.int32, (NP, 1), 0)
            tau, tie_i = _threshold(keys, ridx, K1)
            sm[0] = tau
            sm[1] = tie_i

        tau = sm[0]
        tie_i = sm[1]
        sc = sc_all_ref[pl.ds(i * RC, RC), :]
        key = _sortable(sc)
        rid = i * RC + lax.broadcasted_iota(jnp.int32, (RC, 1), 0)
        m = ((key > tau) | ((key == tau) & (rid <= tie_i))).astype(jnp.float32)
        gate = jnp.tanh(sc) * m
        xbn = jnp.maximum(
            _bn_apply(x_ref[...], st_ref[...], g_ref[...], b_ref[...], N), 0.0)
        xp = xbn * gate
        for cb in range(8):
            xp_ref[cb] = xp[:, cb * FB:(cb + 1) * FB]
        mask_ref[...] = m
        @pl.when(i == 0)
        def _():
            x1_ref[...] = jnp.zeros_like(x1_ref)
        x1_ref[...] += jnp.sum(xp, axis=0, keepdims=True)
        @pl.when(i == NP // RC - 1)
        def _():
            x1_ref[...] = x1_ref[...] * (1.0 / K1)

    return pl.pallas_call(
        body,
        grid=(NP // RC,),
        in_specs=[
            pl.BlockSpec((NP, 1), lambda i: (0, 0)),
            pl.BlockSpec((RC, H), lambda i: (i, 0)),
            pl.BlockSpec((2, H), lambda i: (0, 0)),
            pl.BlockSpec((1, H), lambda i: (0, 0)),
            pl.BlockSpec((1, H), lambda i: (0, 0)),
        ],
        out_specs=[
            pl.BlockSpec((8, RC, FB), lambda i: (0, i, 0)),
            pl.BlockSpec((RC, 1), lambda i: (i, 0)),
            pl.BlockSpec((1, H), lambda i: (0, 0)),
        ],
        out_shape=[
            jax.ShapeDtypeStruct((8, NP, FB), jnp.float32),
            jax.ShapeDtypeStruct((NP, 1), jnp.float32),
            jax.ShapeDtypeStruct((1, H), jnp.float32),
        ],
        scratch_shapes=[pltpu.SMEM((2,), jnp.int32)],
        compiler_params=pltpu.CompilerParams(vmem_limit_bytes=52428800),
    )(score, x, stats, g1, b1b)


def _k4a1(xp, W2_root, b2):
    def body(xp_ref, wo_ref, b2_ref, y_ref):
        y = jnp.broadcast_to(b2_ref[...], (RA, H))
        for cb in range(8):
            y = y + jnp.dot(xp_ref[cb], wo_ref[pl.ds(cb * FB, FB), :],
                            preferred_element_type=jnp.float32)
        y_ref[...] = y

    return pl.pallas_call(
        body,
        grid=(NP // RA,),
        in_specs=[
            pl.BlockSpec((8, RA, FB), lambda i: (0, i, 0)),
            pl.BlockSpec((H, H), lambda i: (0, 0)),
            pl.BlockSpec((1, H), lambda i: (0, 0)),
        ],
        out_specs=pl.BlockSpec((RA, H), lambda i: (i, 0)),
        out_shape=jax.ShapeDtypeStruct((NP, H), jnp.float32),
        compiler_params=pltpu.CompilerParams(vmem_limit_bytes=52428800),
    )(xp, W2_root, b2)


def _k4a2(agg2, y0, W2_rel, mask):
    def body(a_ref, y0_ref, wr_ref, m_ref, y_ref, st_ref):
        i = pl.program_id(0)
        y = y0_ref[...]
        for cb in range(8):
            y = y + jnp.dot(a_ref[cb], wr_ref[pl.ds(cb * FB, FB), :],
                            preferred_element_type=jnp.float32)
        y_ref[...] = y
        ym = y * m_ref[...]
        @pl.when(i == 0)
        def _():
            st_ref[...] = jnp.zeros_like(st_ref)
        st_ref[0:1, :] += jnp.sum(ym, axis=0, keepdims=True)
        st_ref[1:2, :] += jnp.sum(y * ym, axis=0, keepdims=True)

    return pl.pallas_call(
        body,
        grid=(NP // RA,),
        in_specs=[
            pl.BlockSpec((8, RA, FB), lambda i: (0, i, 0)),
            pl.BlockSpec((RA, H), lambda i: (i, 0)),
            pl.BlockSpec((H, H), lambda i: (0, 0)),
            pl.BlockSpec((RA, 1), lambda i: (i, 0)),
        ],
        out_specs=[
            pl.BlockSpec((RA, H), lambda i: (i, 0)),
            pl.BlockSpec((2, H), lambda i: (0, 0)),
        ],
        out_shape=[
            jax.ShapeDtypeStruct((NP, H), jnp.float32),
            jax.ShapeDtypeStruct((2, H), jnp.float32),
        ],
        compiler_params=pltpu.CompilerParams(vmem_limit_bytes=52428800),
    )(agg2, y0, W2_rel, mask)


def _k4b(y, stats2, g2, b2b, w2p, mask):
    def body(y_ref, st_ref, g_ref, b_ref, w_ref, m_ref, s_ref):
        w = w_ref[...]
        wn = w / jnp.sqrt(jnp.sum(w * w))
        ybn = jnp.maximum(
            _bn_apply(y_ref[...], st_ref[...], g_ref[...], b_ref[...], K1), 0.0)
        sc = jnp.sum(ybn * wn, axis=1, keepdims=True)
        s_ref[...] = jnp.where(m_ref[...] > 0, sc, NEG)

    return pl.pallas_call(
        body,
        grid=(NBLK,),
        in_specs=[
            pl.BlockSpec((R, H), lambda i: (i, 0)),
            pl.BlockSpec((2, H), lambda i: (0, 0)),
            pl.BlockSpec((1, H), lambda i: (0, 0)),
            pl.BlockSpec((1, H), lambda i: (0, 0)),
            pl.BlockSpec((1, H), lambda i: (0, 0)),
            pl.BlockSpec((R, 1), lambda i: (i, 0)),
        ],
        out_specs=pl.BlockSpec((R, 1), lambda i: (i, 0)),
        out_shape=jax.ShapeDtypeStruct((NP, 1), jnp.float32),
    )(y, stats2, g2, b2b, w2p, mask)


def _k4cd(score2, y, stats2, g2, b2b, x1, Wfc, bfc, Wfc1, bfc1):
    def body(sc_all_ref, y_ref, st_ref, g_ref, b_ref, x1_ref,
             wfc_ref, bfc_ref, wfc1_ref, bfc1_ref, out_ref, x2acc, sm):
        i = pl.program_id(0)
        @pl.when(i == 0)
        def _():
            keys = _sortable(sc_all_ref[...])
            ridx = lax.broadcasted_iota(jnp.int32, (NP, 1), 0)
            tau, tie_i = _threshold(keys, ridx, K2)
            sm[0] = tau
            sm[1] = tie_i
            x2acc[...] = jnp.zeros_like(x2acc)

        tau = sm[0]
        tie_i = sm[1]
        sc = sc_all_ref[pl.ds(i * RD, RD), :]
        key = _sortable(sc)
        rid = i * RD + lax.broadcasted_iota(jnp.int32, (RD, 1), 0)
        m2 = ((key > tau) | ((key == tau) & (rid <= tie_i))).astype(jnp.float32)
        gate2 = jnp.tanh(sc) * m2
        ybn = jnp.maximum(
            _bn_apply(y_ref[...], st_ref[...], g_ref[...], b_ref[...], K1), 0.0)
        x2acc[...] += jnp.sum(ybn * gate2, axis=0, keepdims=True)

        @pl.when(i == NP // RD - 1)
        def _():
            xf = x1_ref[...] + x2acc[...] * (1.0 / K2)
            h = jnp.maximum(
                jnp.dot(xf, wfc_ref[...], preferred_element_type=jnp.float32)
                + bfc_ref[...], 0.0)
            out_ref[...] = (jnp.dot(h, wfc1_ref[...],
                                    preferred_element_type=jnp.float32)
                            + bfc1_ref[...])

    return pl.pallas_call(
        body,
        grid=(NP // RD,),
        in_specs=[
            pl.BlockSpec((NP, 1), lambda i: (0, 0)),
            pl.BlockSpec((RD, H), lambda i: (i, 0)),
            pl.BlockSpec((2, H), lambda i: (0, 0)),
            pl.BlockSpec((1, H), lambda i: (0, 0)),
            pl.BlockSpec((1, H), lambda i: (0, 0)),
            pl.BlockSpec((1, H), lambda i: (0, 0)),
            pl.BlockSpec((H, 512), lambda i: (0, 0)),
            pl.BlockSpec((1, 512), lambda i: (0, 0)),
            pl.BlockSpec((512, 5), lambda i: (0, 0)),
            pl.BlockSpec((1, 5), lambda i: (0, 0)),
        ],
        out_specs=pl.BlockSpec((1, 5), lambda i: (0, 0)),
        out_shape=jax.ShapeDtypeStruct((1, 5), jnp.float32),
        scratch_shapes=[pltpu.VMEM((1, H), jnp.float32),
                        pltpu.SMEM((2,), jnp.int32)],
        compiler_params=pltpu.CompilerParams(vmem_limit_bytes=52428800),
    )(score2, y, stats2, g2, b2b, x1, Wfc, bfc, Wfc1, bfc1)


def kernel(node, edge_index, batch, W1_rel, b1_rel, W1_root, bn1_g, bn1_b, pool1_w, W2_rel, b2_rel, W2_root, bn2_g, bn2_b, pool2_w, Wfc, bfc, Wfc1, bfc1):
    pad_idx = jnp.asarray(N + (np.arange(EP - E) % 128), dtype=jnp.int32)
    src = jnp.concatenate([edge_index[0], pad_idx])
    dst = jnp.concatenate([edge_index[1], pad_idx])
    node_pad = jnp.pad(node, ((0, NP - N), (0, 0)))

    parts = _seg_sum_128_sc(node_pad, src, dst)

    b1 = b1_rel.reshape(1, H)
    x, stats = _k2a(parts, node_pad, W1_rel, W1_root, b1)
    score = _k2b(x, stats, bn1_g.reshape(1, H), bn1_b.reshape(1, H),
                 pool1_w.reshape(1, H))
    xp, mask, x1 = _k2cd(score, x, stats, bn1_g.reshape(1, H),
                         bn1_b.reshape(1, H))

    agg2_flat = _seg_sum_1024_sc(xp.reshape(8 * NP, FB), src, dst,
                                 mask.reshape(NP))
    agg2 = agg2_flat.reshape(8, NP, FB)

    y0 = _k4a1(xp, W2_root, b2_rel.reshape(1, H))
    y, stats2 = _k4a2(agg2, y0, W2_rel, mask)
    score2 = _k4b(y, stats2, bn2_g.reshape(1, H), bn2_b.reshape(1, H),
                  pool2_w.reshape(1, H), mask)
    out = _k4cd(score2, y, stats2, bn2_g.reshape(1, H), bn2_b.reshape(1, H),
                x1, Wfc, bfc.reshape(1, 512), Wfc1, bfc1.reshape(1, 5))
    return out
